# baseline (speedup 1.0000x reference)
import functools

import jax
import jax.numpy as jnp
from jax import lax
from jax.experimental import pallas as pl
from jax.experimental.pallas import tpu as pltpu
from jax.experimental.pallas import tpu_sc as plsc

_NUM_WORKERS = 32
_CHUNK = 256
_NBUF = 4
_LOOKAHEAD = 2


def _make_emb_kernel(n_total, n_chunks, d):
    mesh = plsc.VectorSubcoreMesh(core_axis_name="c", subcore_axis_name="s")
    per_w = n_chunks * _CHUNK
    slack = _NBUF - _LOOKAHEAD
    assert slack >= 1 and n_chunks % _NBUF == 0 and n_chunks >= 2 * _NBUF

    @functools.partial(
        pl.kernel,
        mesh=mesh,
        out_type=jax.ShapeDtypeStruct((n_total, d), jnp.float32),
        scratch_types=[
            pltpu.VMEM((n_chunks, _CHUNK), jnp.int32),
            pltpu.VMEM((_NBUF, _CHUNK, d), jnp.float32),
            pltpu.SemaphoreType.DMA((_NBUF,)),
            pltpu.SemaphoreType.DMA((_NBUF,)),
        ],
        compiler_params=pltpu.CompilerParams(use_tc_tiling_on_sc=False),
    )
    def emb(x_hbm, tab_hbm, out_hbm, idx_v, rows_v, gsem, osem):
        wid = lax.axis_index("s") * 2 + lax.axis_index("c")
        base = wid * per_w
        pltpu.sync_copy(x_hbm.at[wid], idx_v)

        def fire_gather(j, b):
            pltpu.async_copy(tab_hbm.at[idx_v.at[j]], rows_v.at[b], gsem.at[b])

        def wait_gather(b):
            pltpu.make_async_copy(
                tab_hbm.at[idx_v.at[0]], rows_v.at[b], gsem.at[b]).wait()

        def fire_out(j, b):
            pltpu.async_copy(
                rows_v.at[b], out_hbm.at[pl.ds(base + j * _CHUNK, _CHUNK)],
                osem.at[b])

        def wait_out(b):
            pltpu.make_async_copy(
                rows_v.at[b], out_hbm.at[pl.ds(base, _CHUNK)], osem.at[b]).wait()

        for j in range(_LOOKAHEAD):
            fire_gather(j, j % _NBUF)
        for j in range(slack):
            wait_gather(j % _NBUF)
            fire_out(j, j % _NBUF)
            fire_gather(j + _LOOKAHEAD, (j + _LOOKAHEAD) % _NBUF)

        def group(k, _):
            g = slack + _NBUF * k
            for u in range(_NBUF):
                j = g + u
                b = (slack + u) % _NBUF
                bb = (slack + u + _LOOKAHEAD) % _NBUF
                wait_out(bb)
                fire_gather(j + _LOOKAHEAD, bb)
                wait_gather(b)
                fire_out(j, b)
            return 0

        n_main = n_chunks - _LOOKAHEAD - slack
        assert n_main % _NBUF == 0
        lax.fori_loop(0, n_main // _NBUF, group, 0)

        for j in range(n_chunks - _LOOKAHEAD, n_chunks):
            wait_gather(j % _NBUF)
            fire_out(j, j % _NBUF)
        for b in range(_NBUF):
            wait_out(b)

    return emb


def kernel(x, table):
    b, s = x.shape
    v, d = table.shape
    n = b * s
    assert n % (_NUM_WORKERS * _CHUNK) == 0
    n_chunks = n // (_NUM_WORKERS * _CHUNK)
    x3 = x.reshape(_NUM_WORKERS, n_chunks, _CHUNK).astype(jnp.int32)
    out = _make_emb_kernel(n, n_chunks, 16)(x3, table.reshape(2 * v, 16))
    return out.reshape(b, s, 16)


# speedup vs baseline: 1.0097x; 1.0097x over previous
import functools

import jax
import jax.numpy as jnp
from jax import lax
from jax.experimental import pallas as pl
from jax.experimental.pallas import tpu as pltpu
from jax.experimental.pallas import tpu_sc as plsc

_NUM_WORKERS = 32
_CHUNK = 256
_NBUF = 4
_LOOKAHEAD = 2


def _make_emb_kernel(n_total, n_chunks, d):
    mesh = plsc.VectorSubcoreMesh(core_axis_name="c", subcore_axis_name="s")
    per_w = n_chunks * _CHUNK
    slack = _NBUF - _LOOKAHEAD
    assert slack >= 1 and n_chunks % _NBUF == 0 and n_chunks >= 2 * _NBUF

    @functools.partial(
        pl.kernel,
        mesh=mesh,
        out_type=jax.ShapeDtypeStruct((n_total, d), jnp.float32),
        scratch_types=[
            pltpu.VMEM((n_chunks, _CHUNK), jnp.int32),
            pltpu.VMEM((_NBUF, _CHUNK, d), jnp.float32),
            pltpu.SemaphoreType.DMA((_NBUF,)),
            pltpu.SemaphoreType.DMA((_NBUF,)),
        ],
        compiler_params=pltpu.CompilerParams(use_tc_tiling_on_sc=False),
    )
    def emb(x_hbm, tab_hbm, out_hbm, idx_v, rows_v, gsem, osem):
        wid = lax.axis_index("s") * 2 + lax.axis_index("c")
        base = wid * per_w
        pltpu.sync_copy(x_hbm.at[wid], idx_v)

        def fire_gather(j, b):
            pltpu.async_copy(tab_hbm.at[idx_v.at[j]], rows_v.at[b], gsem.at[b])

        def wait_gather(b):
            pltpu.make_async_copy(
                tab_hbm.at[idx_v.at[0]], rows_v.at[b], gsem.at[b]).wait()

        def fire_out(j, b):
            pltpu.async_copy(
                rows_v.at[b], out_hbm.at[pl.ds(base + j * _CHUNK, _CHUNK)],
                osem.at[b])

        def wait_out(b):
            pltpu.make_async_copy(
                rows_v.at[b], out_hbm.at[pl.ds(base, _CHUNK)], osem.at[b]).wait()

        for j in range(_LOOKAHEAD):
            fire_gather(j, j % _NBUF)
        for j in range(slack):
            wait_gather(j % _NBUF)
            fire_out(j, j % _NBUF)
            fire_gather(j + _LOOKAHEAD, (j + _LOOKAHEAD) % _NBUF)

        def group(k, _):
            g = slack + _NBUF * k
            for u in range(_NBUF):
                j = g + u
                b = (slack + u) % _NBUF
                bb = (slack + u + _LOOKAHEAD) % _NBUF
                wait_out(bb)
                fire_gather(j + _LOOKAHEAD, bb)
                wait_gather(b)
                fire_out(j, b)
            return 0

        n_main = n_chunks - _LOOKAHEAD - slack
        assert n_main % _NBUF == 0
        lax.fori_loop(0, n_main // _NBUF, group, 0)

        for j in range(n_chunks - _LOOKAHEAD, n_chunks):
            wait_gather(j % _NBUF)
            fire_out(j, j % _NBUF)
        for b in range(_NBUF):
            wait_out(b)

    return emb


def kernel(x, table):
    b, s = x.shape
    v, d = table.shape
    n = b * s
    assert n % (_NUM_WORKERS * _CHUNK) == 0
    n_chunks = n // (_NUM_WORKERS * _CHUNK)
    x3 = x.reshape(_NUM_WORKERS, n_chunks, _CHUNK).astype(jnp.int32)
    out = _make_emb_kernel(n, n_chunks, d)(x3, table)
    return out.reshape(b, s, d)
